# Initial kernel scaffold; baseline (speedup 1.0000x reference)
import jax
import jax.numpy as jnp
from jax.experimental import pallas as pl


def kernel(x, table):
    raise NotImplementedError("write your pallas kernel here")


# speedup vs baseline: 1.4592x; 1.4592x over previous
import functools

import jax
import jax.numpy as jnp
from jax import lax
from jax.experimental import pallas as pl
from jax.experimental.pallas import tpu as pltpu
from jax.experimental.pallas import tpu_sc as plsc

_DIM = 32
_IPG = 128
_K = 8
_CHUNK = _K * _IPG


@functools.lru_cache(maxsize=None)
def _make_gather(total: int, dim: int):
    info = plsc.get_sparse_core_info()
    nc, ns = info.num_cores, info.num_subcores
    nw = nc * ns
    assert total % (nw * _CHUNK) == 0
    nb = total // (nw * _CHUNK)
    mesh = plsc.VectorSubcoreMesh(core_axis_name="c", subcore_axis_name="s")

    @functools.partial(
        pl.kernel,
        mesh=mesh,
        out_type=jax.ShapeDtypeStruct((nw, nb, _CHUNK, dim), jnp.float32),
        scratch_types=[
            pltpu.VMEM((_K, _IPG), jnp.int32),
            pltpu.VMEM((_CHUNK, dim), jnp.float32),
            pltpu.SemaphoreType.DMA,
        ],
        compiler_params=pltpu.CompilerParams(use_tc_tiling_on_sc=False),
    )
    def gather(idx_hbm, table_hbm, out_hbm, idx_v, rows_v, sem):
        wid = lax.axis_index("s") * nc + lax.axis_index("c")

        def burst(b, carry):
            pltpu.sync_copy(idx_hbm.at[wid, b], idx_v)
            cps = [
                pltpu.async_copy(
                    table_hbm.at[idx_v.at[j]],
                    rows_v.at[pl.ds(j * _IPG, _IPG)],
                    sem,
                )
                for j in range(_K)
            ]
            for cp in cps:
                cp.wait()
            pltpu.sync_copy(rows_v, out_hbm.at[wid, b])
            return carry

        lax.fori_loop(0, nb, burst, 0)

    return gather


def kernel(x, table):
    b, h = x.shape
    total = b * h
    gather = _make_gather(total, table.shape[1])
    info = plsc.get_sparse_core_info()
    nw = info.num_cores * info.num_subcores
    idx = x.reshape(nw, total // (nw * _CHUNK), _K, _IPG).astype(jnp.int32)
    out = gather(idx, table)
    return out.reshape(b, h, table.shape[1])


# speedup vs baseline: 1.5004x; 1.0282x over previous
import functools

import jax
import jax.numpy as jnp
from jax import lax
from jax.experimental import pallas as pl
from jax.experimental.pallas import tpu as pltpu
from jax.experimental.pallas import tpu_sc as plsc

_IPG = 128
_K = 8
_CHUNK = _K * _IPG


@functools.lru_cache(maxsize=None)
def _make_gather(total: int, dim: int):
    info = plsc.get_sparse_core_info()
    nc, ns = info.num_cores, info.num_subcores
    nw = nc * ns
    assert total % (nw * _CHUNK) == 0
    nb = total // (nw * _CHUNK)
    assert nb % 2 == 1 and nb >= 3
    mesh = plsc.VectorSubcoreMesh(core_axis_name="c", subcore_axis_name="s")

    @functools.partial(
        pl.kernel,
        mesh=mesh,
        out_type=jax.ShapeDtypeStruct((nw, nb, _CHUNK, dim), jnp.float32),
        scratch_types=[
            pltpu.VMEM((nb * _K, _IPG), jnp.int32),
            pltpu.VMEM((_CHUNK, dim), jnp.float32),
            pltpu.VMEM((_CHUNK, dim), jnp.float32),
            pltpu.SemaphoreType.DMA,
            pltpu.SemaphoreType.DMA,
            pltpu.SemaphoreType.DMA,
            pltpu.SemaphoreType.DMA,
        ],
        compiler_params=pltpu.CompilerParams(use_tc_tiling_on_sc=False),
    )
    def gather(idx_hbm, table_hbm, out_hbm, idx_v, rows0, rows1, g0, g1,
               w0, w1):
        wid = lax.axis_index("s") * nc + lax.axis_index("c")
        rows = (rows0, rows1)
        g_sem = (g0, g1)
        w_sem = (w0, w1)

        def fire(cur, buf, sem):
            for j in range(_K):
                pltpu.async_copy(
                    table_hbm.at[idx_v.at[cur * _K + j]],
                    buf.at[pl.ds(j * _IPG, _IPG)],
                    sem,
                )

        def drain_gather(b):
            pltpu.make_async_copy(out_hbm.at[wid, 0], rows[b], g_sem[b]).wait()

        def drain_wb(b):
            pltpu.make_async_copy(rows[b], out_hbm.at[wid, 0], w_sem[b]).wait()

        pltpu.sync_copy(idx_hbm.at[wid], idx_v)

        fire(0, rows[0], g_sem[0])

        def body(g, carry):
            for b in (0, 1):
                cur = 1 + 2 * g + b
                cb = 1 - b
                pb = b

                @pl.when(cur >= 2)
                def _():
                    drain_wb(cb)
                fire(cur, rows[cb], g_sem[cb])
                drain_gather(pb)
                pltpu.async_copy(rows[pb], out_hbm.at[wid, cur - 1], w_sem[pb])
            return carry

        lax.fori_loop(0, (nb - 1) // 2, body, 0)

        drain_gather(0)
        pltpu.async_copy(rows[0], out_hbm.at[wid, nb - 1], w_sem[0])
        drain_wb(1)
        drain_wb(0)

    return gather


def kernel(x, table):
    b, h = x.shape
    total = b * h
    gather = _make_gather(total, table.shape[1])
    info = plsc.get_sparse_core_info()
    nw = info.num_cores * info.num_subcores
    idx = x.reshape(nw, total // (nw * _IPG), _IPG).astype(jnp.int32)
    out = gather(idx, table)
    return out.reshape(b, h, table.shape[1])
